# Initial kernel scaffold; baseline (speedup 1.0000x reference)
import jax
import jax.numpy as jnp
from jax.experimental import pallas as pl


def kernel(x, w_down_W, expert_pos_embed, global_proj_W, global_proj_b, in_proj_W, in_proj_b, out_proj_W, out_proj_b, ln_gamma, ln_beta, scorer_W, scorer_b, mlp_W1, mlp_b1, mlp_W2, mlp_b2, w_up):
    raise NotImplementedError("write your pallas kernel here")


# speedup vs baseline: 2.5937x; 2.5937x over previous
import numpy as np
import jax
import jax.numpy as jnp
from jax.experimental import pallas as pl
from jax.experimental.pallas import tpu as pltpu

D_MODEL = 1024
NE = 8
DL = 64
NH = 4
HD = 16
L = 9
B = 2
S = 2048
N = B * S
T = 512
NB = N // T
BPB = NB // B

_INV_SQRT2 = 0.7071067811865476


def _gelu(x):
    return 0.5 * x * (1.0 + jax.lax.erf(x * _INV_SQRT2))


def _b(t):
    return t.astype(jnp.bfloat16)


def _r32(t):
    return t.astype(jnp.bfloat16).astype(jnp.float32)


def _k1(x_ref, wdt_ref, ef_ref, xsum_ref):
    xb = x_ref[...]
    ef_ref[...] = jnp.dot(_b(xb), _b(wdt_ref[...]),
                          preferred_element_type=jnp.float32)
    xsum_ref[...] = jnp.sum(xb, axis=0, keepdims=True).reshape(1, 1, D_MODEL)


def _k2(ef_ref, xsum_ref, gpt_ref, pos_ref, inwt_ref, inb_ref, outwt_ref,
        outb_ref, gam_ref, bet_ref, scw_ref, scb_ref, gpb_ref, w1t_ref,
        b1_ref, w2t_ref, b2_ref, wup_ref, seg_ref, exp4_ref, exp8_ref,
        out_ref, aux_ref, accp_ref, accl_ref):
    i = pl.program_id(0)
    b = i // BPB

    ef = ef_ref[...]
    efp = ef + pos_ref[...]

    xrows = xsum_ref[...][:, 0, :]
    riota = jax.lax.broadcasted_iota(jnp.int32, (NB, 1), 0)
    rmask = (riota // BPB == b).astype(jnp.float32)
    xmean = jnp.sum(xrows * rmask, axis=0, keepdims=True) / S
    gc = jnp.dot(_b(xmean), _b(gpt_ref[...]),
                 preferred_element_type=jnp.float32) + gpb_ref[...]
    qkvg = jnp.dot(_b(gc), _b(inwt_ref[...]),
                   preferred_element_type=jnp.float32) + inb_ref[...]
    h = _gelu(jnp.dot(_b(gc), _b(w1t_ref[...]),
                      preferred_element_type=jnp.float32) + b1_ref[...])
    gb = jnp.dot(_b(h), _b(w2t_ref[...]),
                 preferred_element_type=jnp.float32) + b2_ref[...]

    inwt = _b(inwt_ref[...])
    qkv = [jnp.dot(_b(efp[:, e * DL:(e + 1) * DL]), inwt,
                   preferred_element_type=jnp.float32) + inb_ref[...]
           for e in range(NE)]
    kg = jnp.broadcast_to(qkvg[:, DL:2 * DL], (T, DL))
    vg = jnp.broadcast_to(qkvg[:, 2 * DL:3 * DL], (T, DL))
    k_row = _r32(jnp.concatenate([kg] + [q[:, DL:2 * DL] for q in qkv], axis=1))
    v_row = _r32(jnp.concatenate([vg] + [q[:, 2 * DL:3 * DL] for q in qkv], axis=1))

    outwt = _b(outwt_ref[...])
    scwt = _b(scw_ref[...])
    logits_cols = []
    for e in range(NE):
        q_e = _r32(qkv[e][:, :DL])
        p = jnp.concatenate([q_e] * L, axis=1) * k_row
        s = jnp.dot(p, seg_ref[...], preferred_element_type=jnp.float32, precision=jax.lax.Precision.HIGHEST)
        m = s[:, 0:NH]
        for j in range(1, L):
            m = jnp.maximum(m, s[:, NH * j:NH * (j + 1)])
        es = [jnp.exp(s[:, NH * j:NH * (j + 1)] - m) for j in range(L)]
        den = es[0]
        for j in range(1, L):
            den = den + es[j]
        o = jnp.zeros((T, DL), dtype=jnp.float32)
        for j in range(L):
            a = _r32(es[j] / den)
            o = o + jnp.dot(a, exp4_ref[...],
                            preferred_element_type=jnp.float32, precision=jax.lax.Precision.HIGHEST) \
                    * v_row[:, DL * j:DL * (j + 1)]
        proj = jnp.dot(_b(o), outwt,
                       preferred_element_type=jnp.float32) + outb_ref[...]
        v_res = proj + efp[:, e * DL:(e + 1) * DL]
        mu = jnp.mean(v_res, axis=-1, keepdims=True)
        var = jnp.mean((v_res - mu) ** 2, axis=-1, keepdims=True)
        inter = (v_res - mu) * jax.lax.rsqrt(var + 1e-5) * gam_ref[...] + bet_ref[...]
        logit = jnp.dot(_b(inter), scwt,
                        preferred_element_type=jnp.float32) + scb_ref[0, 0]
        logits_cols.append(logit)

    final_logits = jnp.concatenate(logits_cols, axis=1) + gb

    mx = jnp.max(final_logits, axis=-1, keepdims=True)
    ex = jnp.exp(final_logits - mx)
    probs = ex / jnp.sum(ex, axis=-1, keepdims=True)
    iota = jax.lax.broadcasted_iota(jnp.int32, (T, NE), 1)
    big = jnp.int32(NE + 1)
    m1 = jnp.max(probs, axis=-1, keepdims=True)
    idx1 = jnp.min(jnp.where(probs == m1, iota, big), axis=-1, keepdims=True)
    oh1 = iota == idx1
    p2 = jnp.where(oh1, -1.0, probs)
    m2 = jnp.max(p2, axis=-1, keepdims=True)
    idx2 = jnp.min(jnp.where(p2 == m2, iota, big), axis=-1, keepdims=True)
    oh2 = iota == idx2
    denom = m1 + m2
    comb = jnp.where(oh1, m1 / denom, 0.0) + jnp.where(oh2, m2 / denom, 0.0)
    load = oh1.astype(jnp.float32) + oh2.astype(jnp.float32)

    @pl.when(i == 0)
    def _():
        accp_ref[...] = jnp.zeros_like(accp_ref)
        accl_ref[...] = jnp.zeros_like(accl_ref)

    accp_ref[...] += jnp.sum(probs, axis=0, keepdims=True)
    accl_ref[...] += jnp.sum(load, axis=0, keepdims=True)

    @pl.when(i == NB - 1)
    def _():
        aux = (NE / (N * N)) * jnp.sum(accp_ref[...] * accl_ref[...])
        aux_ref[...] = jnp.full((1, 1), aux, dtype=jnp.float32)

    act = _r32(_gelu(ef))
    wa = act * jnp.dot(comb, exp8_ref[...],
                       preferred_element_type=jnp.float32, precision=jax.lax.Precision.HIGHEST)
    out_ref[...] = jnp.dot(wa, _r32(wup_ref[...]),
                           preferred_element_type=jnp.float32, precision=jax.lax.Precision.HIGHEST)


def kernel(x, w_down_W, expert_pos_embed, global_proj_W, global_proj_b,
           in_proj_W, in_proj_b, out_proj_W, out_proj_b, ln_gamma, ln_beta,
           scorer_W, scorer_b, mlp_W1, mlp_b1, mlp_W2, mlp_b2, w_up):
    x_flat = x.reshape(N, D_MODEL)

    ef, xsum = pl.pallas_call(
        _k1,
        grid=(NB,),
        in_specs=[
            pl.BlockSpec((T, D_MODEL), lambda i: (i, 0)),
            pl.BlockSpec((D_MODEL, NE * DL), lambda i: (0, 0)),
        ],
        out_specs=[
            pl.BlockSpec((T, NE * DL), lambda i: (i, 0)),
            pl.BlockSpec((1, 1, D_MODEL), lambda i: (i, 0, 0)),
        ],
        out_shape=[
            jax.ShapeDtypeStruct((N, NE * DL), jnp.float32),
            jax.ShapeDtypeStruct((NB, 1, D_MODEL), jnp.float32),
        ],
    )(x_flat, w_down_W.T)

    seg = np.zeros((L * DL, L * NH), dtype=np.float32)
    for j in range(L):
        for hh in range(NH):
            seg[j * DL + hh * HD:(j * DL + (hh + 1) * HD), j * NH + hh] = 0.25
    exp4 = np.zeros((NH, DL), dtype=np.float32)
    for hh in range(NH):
        exp4[hh, hh * HD:(hh + 1) * HD] = 1.0
    exp8 = np.zeros((NE, NE * DL), dtype=np.float32)
    for e in range(NE):
        exp8[e, e * DL:(e + 1) * DL] = 1.0

    const2 = lambda i: (0, 0)
    small = lambda a: pl.BlockSpec(a.shape, const2)

    pos = expert_pos_embed.reshape(1, NE * DL)
    args = (
        ef, xsum, global_proj_W.T, pos,
        in_proj_W.T, in_proj_b.reshape(1, -1),
        out_proj_W.T, out_proj_b.reshape(1, -1),
        ln_gamma.reshape(1, -1), ln_beta.reshape(1, -1),
        scorer_W.reshape(-1, 1), scorer_b.reshape(1, 1),
        global_proj_b.reshape(1, -1),
        mlp_W1.T, mlp_b1.reshape(1, -1),
        mlp_W2.T, mlp_b2.reshape(1, -1),
        w_up.reshape(NE * DL, D_MODEL),
        jnp.asarray(seg), jnp.asarray(exp4), jnp.asarray(exp8),
    )

    in_specs = [
        pl.BlockSpec((T, NE * DL), lambda i: (i, 0)),
        pl.BlockSpec((NB, 1, D_MODEL), lambda i: (0, 0, 0)),
    ] + [small(a) for a in args[2:]]

    out, aux = pl.pallas_call(
        _k2,
        grid=(NB,),
        in_specs=in_specs,
        out_specs=[
            pl.BlockSpec((T, D_MODEL), lambda i: (i, 0)),
            pl.BlockSpec((1, 1), const2),
        ],
        out_shape=[
            jax.ShapeDtypeStruct((N, D_MODEL), jnp.float32),
            jax.ShapeDtypeStruct((1, 1), jnp.float32),
        ],
        scratch_shapes=[
            pltpu.VMEM((1, NE), jnp.float32),
            pltpu.VMEM((1, NE), jnp.float32),
        ],
    )(*args)

    return out.reshape(B, S, D_MODEL), aux[0, 0]


# speedup vs baseline: 3.0901x; 1.1914x over previous
import numpy as np
import jax
import jax.numpy as jnp
from jax.experimental import pallas as pl
from jax.experimental.pallas import tpu as pltpu

D_MODEL = 1024
NE = 8
DL = 64
NH = 4
HD = 16
L = 9
B = 2
S = 2048
N = B * S
T = 512
NB = N // T
BPB = NB // B

_INV_SQRT2 = 0.7071067811865476


def _gelu(x):
    return 0.5 * x * (1.0 + jax.lax.erf(x * _INV_SQRT2))


def _b(t):
    return t.astype(jnp.bfloat16)


def _r32(t):
    return t.astype(jnp.bfloat16).astype(jnp.float32)


def _k1(x_ref, wdt_ref, ef_ref, xsum_ref):
    xb = x_ref[...]
    ef_ref[...] = jnp.dot(_b(xb), _b(wdt_ref[...]),
                          preferred_element_type=jnp.float32)
    xsum_ref[...] = jnp.sum(xb, axis=0, keepdims=True).reshape(1, 1, D_MODEL)


def _k2(ef_ref, xsum_ref, gpt_ref, pos_ref, inwt_ref, inb_ref, outwt_ref,
        outb_ref, gam_ref, bet_ref, scw_ref, scb_ref, gpb_ref, w1t_ref,
        b1_ref, w2t_ref, b2_ref, wup_ref, seg_ref, exp4_ref, exp8_ref,
        out_ref, aux_ref, accp_ref, accl_ref):
    i = pl.program_id(0)
    b = i // BPB

    ef = ef_ref[...]
    efp = ef + pos_ref[...]

    xrows = xsum_ref[...][:, 0, :]
    riota = jax.lax.broadcasted_iota(jnp.int32, (NB, 1), 0)
    rmask = (riota // BPB == b).astype(jnp.float32)
    xmean = jnp.sum(xrows * rmask, axis=0, keepdims=True) / S
    gc = jnp.dot(_b(xmean), _b(gpt_ref[...]),
                 preferred_element_type=jnp.float32) + gpb_ref[...]
    qkvg = jnp.dot(_b(gc), _b(inwt_ref[...]),
                   preferred_element_type=jnp.float32) + inb_ref[...]
    h = _gelu(jnp.dot(_b(gc), _b(w1t_ref[...]),
                      preferred_element_type=jnp.float32) + b1_ref[...])
    gb = jnp.dot(_b(h), _b(w2t_ref[...]),
                 preferred_element_type=jnp.float32) + b2_ref[...]

    inwt = _b(inwt_ref[...])
    qkv = [jnp.dot(_b(efp[:, e * DL:(e + 1) * DL]), inwt,
                   preferred_element_type=jnp.float32) + inb_ref[...]
           for e in range(NE)]
    kg = jnp.broadcast_to(qkvg[:, DL:2 * DL], (T, DL))
    vg = jnp.broadcast_to(qkvg[:, 2 * DL:3 * DL], (T, DL))
    k_row = _r32(jnp.concatenate([kg] + [q[:, DL:2 * DL] for q in qkv], axis=1))
    v_row = _r32(jnp.concatenate([vg] + [q[:, 2 * DL:3 * DL] for q in qkv], axis=1))

    outwt = _b(outwt_ref[...])
    scwt = _b(scw_ref[...])
    logits_cols = []
    for e in range(NE):
        q_e = _r32(qkv[e][:, :DL])
        p = jnp.concatenate([q_e] * L, axis=1) * k_row
        s = jnp.dot(p, seg_ref[...], preferred_element_type=jnp.float32,
                    precision=jax.lax.Precision.HIGHEST)
        m = s[:, 0:NH]
        for j in range(1, L):
            m = jnp.maximum(m, s[:, NH * j:NH * (j + 1)])
        es = [jnp.exp(s[:, NH * j:NH * (j + 1)] - m) for j in range(L)]
        den = es[0]
        for j in range(1, L):
            den = den + es[j]
        o = jnp.zeros((T, DL), dtype=jnp.float32)
        for j in range(L):
            a = _r32(es[j] / den)
            o = o + jnp.dot(_b(a), _b(exp4_ref[...]),
                            preferred_element_type=jnp.float32) \
                    * v_row[:, DL * j:DL * (j + 1)]
        proj = jnp.dot(_b(o), outwt,
                       preferred_element_type=jnp.float32) + outb_ref[...]
        v_res = proj + efp[:, e * DL:(e + 1) * DL]
        mu = jnp.mean(v_res, axis=-1, keepdims=True)
        var = jnp.mean((v_res - mu) ** 2, axis=-1, keepdims=True)
        inter = (v_res - mu) * jax.lax.rsqrt(var + 1e-5) * gam_ref[...] + bet_ref[...]
        logit = jnp.dot(_b(inter), scwt,
                        preferred_element_type=jnp.float32) + scb_ref[0, 0]
        logits_cols.append(logit)

    final_logits = jnp.concatenate(logits_cols, axis=1) + gb

    mx = jnp.max(final_logits, axis=-1, keepdims=True)
    ex = jnp.exp(final_logits - mx)
    probs = ex / jnp.sum(ex, axis=-1, keepdims=True)
    iota = jax.lax.broadcasted_iota(jnp.int32, (T, NE), 1)
    big = jnp.int32(NE + 1)
    m1 = jnp.max(probs, axis=-1, keepdims=True)
    idx1 = jnp.min(jnp.where(probs == m1, iota, big), axis=-1, keepdims=True)
    oh1 = iota == idx1
    p2 = jnp.where(oh1, -1.0, probs)
    m2 = jnp.max(p2, axis=-1, keepdims=True)
    idx2 = jnp.min(jnp.where(p2 == m2, iota, big), axis=-1, keepdims=True)
    oh2 = iota == idx2
    denom = m1 + m2
    comb = jnp.where(oh1, m1 / denom, 0.0) + jnp.where(oh2, m2 / denom, 0.0)
    load = oh1.astype(jnp.float32) + oh2.astype(jnp.float32)

    @pl.when(i == 0)
    def _():
        accp_ref[...] = jnp.zeros_like(accp_ref)
        accl_ref[...] = jnp.zeros_like(accl_ref)

    accp_ref[...] += jnp.sum(probs, axis=0, keepdims=True)
    accl_ref[...] += jnp.sum(load, axis=0, keepdims=True)

    @pl.when(i == NB - 1)
    def _():
        aux = (NE / (N * N)) * jnp.sum(accp_ref[...] * accl_ref[...])
        aux_ref[...] = jnp.full((1, 1), aux, dtype=jnp.float32)

    act = _b(_gelu(ef))
    wup = _b(wup_ref[...])
    acc = jnp.zeros((T, D_MODEL), dtype=jnp.float32)
    for e in range(NE):
        z = jnp.dot(act[:, e * DL:(e + 1) * DL], wup[e * DL:(e + 1) * DL, :],
                    preferred_element_type=jnp.float32)
        acc = acc + jnp.broadcast_to(comb[:, e:e + 1], (T, D_MODEL)) * z
    out_ref[...] = acc


def kernel(x, w_down_W, expert_pos_embed, global_proj_W, global_proj_b,
           in_proj_W, in_proj_b, out_proj_W, out_proj_b, ln_gamma, ln_beta,
           scorer_W, scorer_b, mlp_W1, mlp_b1, mlp_W2, mlp_b2, w_up):
    x_flat = x.reshape(N, D_MODEL)

    ef, xsum = pl.pallas_call(
        _k1,
        grid=(NB,),
        in_specs=[
            pl.BlockSpec((T, D_MODEL), lambda i: (i, 0)),
            pl.BlockSpec((D_MODEL, NE * DL), lambda i: (0, 0)),
        ],
        out_specs=[
            pl.BlockSpec((T, NE * DL), lambda i: (i, 0)),
            pl.BlockSpec((1, 1, D_MODEL), lambda i: (i, 0, 0)),
        ],
        out_shape=[
            jax.ShapeDtypeStruct((N, NE * DL), jnp.float32),
            jax.ShapeDtypeStruct((NB, 1, D_MODEL), jnp.float32),
        ],
    )(x_flat, w_down_W.T)

    seg = np.zeros((L * DL, L * NH), dtype=np.float32)
    for j in range(L):
        for hh in range(NH):
            seg[j * DL + hh * HD:(j * DL + (hh + 1) * HD), j * NH + hh] = 0.25
    exp4 = np.zeros((NH, DL), dtype=np.float32)
    for hh in range(NH):
        exp4[hh, hh * HD:(hh + 1) * HD] = 1.0
    exp8 = np.zeros((NE, NE * DL), dtype=np.float32)
    for e in range(NE):
        exp8[e, e * DL:(e + 1) * DL] = 1.0

    const2 = lambda i: (0, 0)
    small = lambda a: pl.BlockSpec(a.shape, const2)

    pos = expert_pos_embed.reshape(1, NE * DL)
    args = (
        ef, xsum, global_proj_W.T, pos,
        in_proj_W.T, in_proj_b.reshape(1, -1),
        out_proj_W.T, out_proj_b.reshape(1, -1),
        ln_gamma.reshape(1, -1), ln_beta.reshape(1, -1),
        scorer_W.reshape(-1, 1), scorer_b.reshape(1, 1),
        global_proj_b.reshape(1, -1),
        mlp_W1.T, mlp_b1.reshape(1, -1),
        mlp_W2.T, mlp_b2.reshape(1, -1),
        w_up.reshape(NE * DL, D_MODEL),
        jnp.asarray(seg), jnp.asarray(exp4), jnp.asarray(exp8),
    )

    in_specs = [
        pl.BlockSpec((T, NE * DL), lambda i: (i, 0)),
        pl.BlockSpec((NB, 1, D_MODEL), lambda i: (0, 0, 0)),
    ] + [small(a) for a in args[2:]]

    out, aux = pl.pallas_call(
        _k2,
        grid=(NB,),
        in_specs=in_specs,
        out_specs=[
            pl.BlockSpec((T, D_MODEL), lambda i: (i, 0)),
            pl.BlockSpec((1, 1), const2),
        ],
        out_shape=[
            jax.ShapeDtypeStruct((N, D_MODEL), jnp.float32),
            jax.ShapeDtypeStruct((1, 1), jnp.float32),
        ],
        scratch_shapes=[
            pltpu.VMEM((1, NE), jnp.float32),
            pltpu.VMEM((1, NE), jnp.float32),
        ],
    )(*args)

    return out.reshape(B, S, D_MODEL), aux[0, 0]


# speedup vs baseline: 4.6287x; 1.4979x over previous
import numpy as np
import jax
import jax.numpy as jnp
from jax.experimental import pallas as pl
from jax.experimental.pallas import tpu as pltpu

D_MODEL = 1024
NE = 8
DL = 64
NH = 4
HD = 16
L = 9
B = 2
S = 2048
N = B * S
T = 512
NB = N // T
BPB = NB // B

_INV_SQRT2 = 0.7071067811865476


def _gelu(x):
    return 0.5 * x * (1.0 + jax.lax.erf(x * _INV_SQRT2))


def _b(t):
    return t.astype(jnp.bfloat16)


def _r32(t):
    return t.astype(jnp.bfloat16).astype(jnp.float32)


def _k1(x_ref, wdt_ref, ef_ref, xsum_ref):
    xb = x_ref[...]
    ef_ref[...] = jnp.dot(_b(xb), _b(wdt_ref[...]),
                          preferred_element_type=jnp.float32)
    xsum_ref[...] = jnp.sum(xb, axis=0, keepdims=True).reshape(1, 1, D_MODEL)


def _k2(ef_ref, xsum_ref, gpt_ref, pos_ref, inwt_ref, inb_ref, outwt_ref,
        outb_ref, gam_ref, bet_ref, scw_ref, scb_ref, gpb_ref, w1t_ref,
        b1_ref, w2t_ref, b2_ref, wup_ref, seg_ref, exp4_ref, exp8_ref,
        out_ref, aux_ref, accp_ref, accl_ref):
    i = pl.program_id(0)
    b = i // BPB

    ef = ef_ref[...]
    efp = ef + pos_ref[...]

    xrows = xsum_ref[...][:, 0, :]
    riota = jax.lax.broadcasted_iota(jnp.int32, (NB, 1), 0)
    rmask = (riota // BPB == b).astype(jnp.float32)
    xmean = jnp.sum(xrows * rmask, axis=0, keepdims=True) / S
    gc = jnp.dot(_b(xmean), _b(gpt_ref[...]),
                 preferred_element_type=jnp.float32) + gpb_ref[...]
    qkvg = jnp.dot(_b(gc), _b(inwt_ref[...]),
                   preferred_element_type=jnp.float32) + inb_ref[...]
    h = _gelu(jnp.dot(_b(gc), _b(w1t_ref[...]),
                      preferred_element_type=jnp.float32) + b1_ref[...])
    gb = jnp.dot(_b(h), _b(w2t_ref[...]),
                 preferred_element_type=jnp.float32) + b2_ref[...]

    inwt = _b(inwt_ref[...])
    qkv = [jnp.dot(_b(efp[:, e * DL:(e + 1) * DL]), inwt,
                   preferred_element_type=jnp.float32) + inb_ref[...]
           for e in range(NE)]
    kg = jnp.broadcast_to(qkvg[:, DL:2 * DL], (T, DL))
    vg = jnp.broadcast_to(qkvg[:, 2 * DL:3 * DL], (T, DL))
    k_row = _r32(jnp.concatenate([kg] + [q[:, DL:2 * DL] for q in qkv], axis=1))
    v_row = _r32(jnp.concatenate([vg] + [q[:, 2 * DL:3 * DL] for q in qkv], axis=1))

    outwt = _b(outwt_ref[...])
    scwt = _b(scw_ref[...])
    logits_cols = []
    for e in range(NE):
        q_e = _r32(qkv[e][:, :DL])
        p = jnp.concatenate([q_e] * L, axis=1) * k_row
        p_hi = _b(p)
        p_lo = _b(p - p_hi.astype(jnp.float32))
        segb = _b(seg_ref[...])
        s = (jnp.dot(p_hi, segb, preferred_element_type=jnp.float32)
             + jnp.dot(p_lo, segb, preferred_element_type=jnp.float32))
        m = s[:, 0:NH]
        for j in range(1, L):
            m = jnp.maximum(m, s[:, NH * j:NH * (j + 1)])
        es = [jnp.exp(s[:, NH * j:NH * (j + 1)] - m) for j in range(L)]
        den = es[0]
        for j in range(1, L):
            den = den + es[j]
        o = jnp.zeros((T, DL), dtype=jnp.float32)
        for j in range(L):
            a = _r32(es[j] / den)
            o = o + jnp.dot(_b(a), _b(exp4_ref[...]),
                            preferred_element_type=jnp.float32) \
                    * v_row[:, DL * j:DL * (j + 1)]
        proj = jnp.dot(_b(o), outwt,
                       preferred_element_type=jnp.float32) + outb_ref[...]
        v_res = proj + efp[:, e * DL:(e + 1) * DL]
        mu = jnp.mean(v_res, axis=-1, keepdims=True)
        var = jnp.mean((v_res - mu) ** 2, axis=-1, keepdims=True)
        inter = (v_res - mu) * jax.lax.rsqrt(var + 1e-5) * gam_ref[...] + bet_ref[...]
        logit = jnp.dot(_b(inter), scwt,
                        preferred_element_type=jnp.float32) + scb_ref[0, 0]
        logits_cols.append(logit)

    final_logits = jnp.concatenate(logits_cols, axis=1) + gb

    mx = jnp.max(final_logits, axis=-1, keepdims=True)
    ex = jnp.exp(final_logits - mx)
    probs = ex / jnp.sum(ex, axis=-1, keepdims=True)
    iota = jax.lax.broadcasted_iota(jnp.int32, (T, NE), 1)
    big = jnp.int32(NE + 1)
    m1 = jnp.max(probs, axis=-1, keepdims=True)
    idx1 = jnp.min(jnp.where(probs == m1, iota, big), axis=-1, keepdims=True)
    oh1 = iota == idx1
    p2 = jnp.where(oh1, -1.0, probs)
    m2 = jnp.max(p2, axis=-1, keepdims=True)
    idx2 = jnp.min(jnp.where(p2 == m2, iota, big), axis=-1, keepdims=True)
    oh2 = iota == idx2
    denom = m1 + m2
    comb = jnp.where(oh1, m1 / denom, 0.0) + jnp.where(oh2, m2 / denom, 0.0)
    load = oh1.astype(jnp.float32) + oh2.astype(jnp.float32)

    @pl.when(i == 0)
    def _():
        accp_ref[...] = jnp.zeros_like(accp_ref)
        accl_ref[...] = jnp.zeros_like(accl_ref)

    accp_ref[...] += jnp.sum(probs, axis=0, keepdims=True)
    accl_ref[...] += jnp.sum(load, axis=0, keepdims=True)

    @pl.when(i == NB - 1)
    def _():
        aux = (NE / (N * N)) * jnp.sum(accp_ref[...] * accl_ref[...])
        aux_ref[...] = jnp.full((1, 1), aux, dtype=jnp.float32)

    act = _b(_gelu(ef))
    wup = _b(wup_ref[...])
    acc = jnp.zeros((T, D_MODEL), dtype=jnp.float32)
    for e in range(NE):
        z = jnp.dot(act[:, e * DL:(e + 1) * DL], wup[e * DL:(e + 1) * DL, :],
                    preferred_element_type=jnp.float32)
        acc = acc + jnp.broadcast_to(comb[:, e:e + 1], (T, D_MODEL)) * z
    out_ref[...] = acc


def kernel(x, w_down_W, expert_pos_embed, global_proj_W, global_proj_b,
           in_proj_W, in_proj_b, out_proj_W, out_proj_b, ln_gamma, ln_beta,
           scorer_W, scorer_b, mlp_W1, mlp_b1, mlp_W2, mlp_b2, w_up):
    x_flat = x.reshape(N, D_MODEL)

    ef, xsum = pl.pallas_call(
        _k1,
        grid=(NB,),
        in_specs=[
            pl.BlockSpec((T, D_MODEL), lambda i: (i, 0)),
            pl.BlockSpec((D_MODEL, NE * DL), lambda i: (0, 0)),
        ],
        out_specs=[
            pl.BlockSpec((T, NE * DL), lambda i: (i, 0)),
            pl.BlockSpec((1, 1, D_MODEL), lambda i: (i, 0, 0)),
        ],
        out_shape=[
            jax.ShapeDtypeStruct((N, NE * DL), jnp.float32),
            jax.ShapeDtypeStruct((NB, 1, D_MODEL), jnp.float32),
        ],
    )(x_flat, w_down_W.T)

    seg = np.zeros((L * DL, L * NH), dtype=np.float32)
    for j in range(L):
        for hh in range(NH):
            seg[j * DL + hh * HD:(j * DL + (hh + 1) * HD), j * NH + hh] = 0.25
    exp4 = np.zeros((NH, DL), dtype=np.float32)
    for hh in range(NH):
        exp4[hh, hh * HD:(hh + 1) * HD] = 1.0
    exp8 = np.zeros((NE, NE * DL), dtype=np.float32)
    for e in range(NE):
        exp8[e, e * DL:(e + 1) * DL] = 1.0

    const2 = lambda i: (0, 0)
    small = lambda a: pl.BlockSpec(a.shape, const2)

    pos = expert_pos_embed.reshape(1, NE * DL)
    args = (
        ef, xsum, global_proj_W.T, pos,
        in_proj_W.T, in_proj_b.reshape(1, -1),
        out_proj_W.T, out_proj_b.reshape(1, -1),
        ln_gamma.reshape(1, -1), ln_beta.reshape(1, -1),
        scorer_W.reshape(-1, 1), scorer_b.reshape(1, 1),
        global_proj_b.reshape(1, -1),
        mlp_W1.T, mlp_b1.reshape(1, -1),
        mlp_W2.T, mlp_b2.reshape(1, -1),
        w_up.reshape(NE * DL, D_MODEL),
        jnp.asarray(seg), jnp.asarray(exp4), jnp.asarray(exp8),
    )

    in_specs = [
        pl.BlockSpec((T, NE * DL), lambda i: (i, 0)),
        pl.BlockSpec((NB, 1, D_MODEL), lambda i: (0, 0, 0)),
    ] + [small(a) for a in args[2:]]

    out, aux = pl.pallas_call(
        _k2,
        grid=(NB,),
        in_specs=in_specs,
        out_specs=[
            pl.BlockSpec((T, D_MODEL), lambda i: (i, 0)),
            pl.BlockSpec((1, 1), const2),
        ],
        out_shape=[
            jax.ShapeDtypeStruct((N, D_MODEL), jnp.float32),
            jax.ShapeDtypeStruct((1, 1), jnp.float32),
        ],
        scratch_shapes=[
            pltpu.VMEM((1, NE), jnp.float32),
            pltpu.VMEM((1, NE), jnp.float32),
        ],
    )(*args)

    return out.reshape(B, S, D_MODEL), aux[0, 0]


# speedup vs baseline: 4.6338x; 1.0011x over previous
import numpy as np
import jax
import jax.numpy as jnp
from jax.experimental import pallas as pl
from jax.experimental.pallas import tpu as pltpu

D_MODEL = 1024
NE = 8
DL = 64
NH = 4
HD = 16
L = 9
B = 2
S = 2048
N = B * S
T = 512
NB = N // T
BPB = NB // B

_INV_SQRT2 = 0.7071067811865476


def _gelu(x):
    return 0.5 * x * (1.0 + jax.lax.erf(x * _INV_SQRT2))


def _b(t):
    return t.astype(jnp.bfloat16)


def _r32(t):
    return t.astype(jnp.bfloat16).astype(jnp.float32)


def _k1(x_ref, wdt_ref, ef_ref, xsum_ref):
    xb = x_ref[...]
    ef_ref[...] = jnp.dot(_b(xb), _b(wdt_ref[...]),
                          preferred_element_type=jnp.float32)
    xsum_ref[...] = jnp.sum(xb, axis=0, keepdims=True).reshape(1, 1, D_MODEL)


def _k2(ef_ref, xsum_ref, gpt_ref, pos_ref, inwt_ref, inb_ref, outwt_ref,
        outb_ref, gam_ref, bet_ref, scw_ref, scb_ref, gpb_ref, w1t_ref,
        b1_ref, w2t_ref, b2_ref, wup_ref, seg_ref, exp4_ref,
        out_ref, aux_ref, accp_ref, accl_ref):
    i = pl.program_id(0)
    b = i // BPB

    ef = ef_ref[...]
    efp = ef + pos_ref[...]

    xrows = xsum_ref[...][:, 0, :]
    riota = jax.lax.broadcasted_iota(jnp.int32, (NB, 1), 0)
    rmask = (riota // BPB == b).astype(jnp.float32)
    xmean = jnp.sum(xrows * rmask, axis=0, keepdims=True) / S
    gc = jnp.dot(_b(xmean), _b(gpt_ref[...]),
                 preferred_element_type=jnp.float32) + gpb_ref[...]
    qkvg = jnp.dot(_b(gc), _b(inwt_ref[...]),
                   preferred_element_type=jnp.float32) + inb_ref[...]
    h = _gelu(jnp.dot(_b(gc), _b(w1t_ref[...]),
                      preferred_element_type=jnp.float32) + b1_ref[...])
    gb = jnp.dot(_b(h), _b(w2t_ref[...]),
                 preferred_element_type=jnp.float32) + b2_ref[...]

    inwt = _b(inwt_ref[...])
    qkv = [jnp.dot(_b(efp[:, e * DL:(e + 1) * DL]), inwt,
                   preferred_element_type=jnp.float32) + inb_ref[...]
           for e in range(NE)]
    kg = jnp.broadcast_to(qkvg[:, DL:2 * DL], (T, DL))
    vg = jnp.broadcast_to(qkvg[:, 2 * DL:3 * DL], (T, DL))
    k_row = _r32(jnp.concatenate([kg] + [q[:, DL:2 * DL] for q in qkv], axis=1))
    v_row = _r32(jnp.concatenate([vg] + [q[:, 2 * DL:3 * DL] for q in qkv], axis=1))

    outwt = _b(outwt_ref[...])
    scwt = _b(scw_ref[...])
    logits_cols = []
    for e in range(NE):
        q_e = _r32(qkv[e][:, :DL])
        p = jnp.concatenate([q_e] * L, axis=1) * k_row
        p_hi = _b(p)
        p_lo = _b(p - p_hi.astype(jnp.float32))
        segb = _b(seg_ref[...])
        s = (jnp.dot(p_hi, segb, preferred_element_type=jnp.float32)
             + jnp.dot(p_lo, segb, preferred_element_type=jnp.float32))
        m = s[:, 0:NH]
        for j in range(1, L):
            m = jnp.maximum(m, s[:, NH * j:NH * (j + 1)])
        es = [jnp.exp(s[:, NH * j:NH * (j + 1)] - m) for j in range(L)]
        den = es[0]
        for j in range(1, L):
            den = den + es[j]
        o = jnp.zeros((T, DL), dtype=jnp.float32)
        for j in range(L):
            a = _r32(es[j] / den)
            o = o + jnp.dot(_b(a), _b(exp4_ref[...]),
                            preferred_element_type=jnp.float32) \
                    * v_row[:, DL * j:DL * (j + 1)]
        proj = jnp.dot(_b(o), outwt,
                       preferred_element_type=jnp.float32) + outb_ref[...]
        v_res = proj + efp[:, e * DL:(e + 1) * DL]
        mu = jnp.mean(v_res, axis=-1, keepdims=True)
        var = jnp.mean((v_res - mu) ** 2, axis=-1, keepdims=True)
        inter = (v_res - mu) * jax.lax.rsqrt(var + 1e-5) * gam_ref[...] + bet_ref[...]
        logit = jnp.dot(_b(inter), scwt,
                        preferred_element_type=jnp.float32) + scb_ref[0, 0]
        logits_cols.append(logit)

    final_logits = jnp.concatenate(logits_cols, axis=1) + gb

    mx = jnp.max(final_logits, axis=-1, keepdims=True)
    ex = jnp.exp(final_logits - mx)
    probs = ex / jnp.sum(ex, axis=-1, keepdims=True)
    iota = jax.lax.broadcasted_iota(jnp.int32, (T, NE), 1)
    big = jnp.int32(NE + 1)
    m1 = jnp.max(probs, axis=-1, keepdims=True)
    idx1 = jnp.min(jnp.where(probs == m1, iota, big), axis=-1, keepdims=True)
    oh1 = iota == idx1
    p2 = jnp.where(oh1, -1.0, probs)
    m2 = jnp.max(p2, axis=-1, keepdims=True)
    idx2 = jnp.min(jnp.where(p2 == m2, iota, big), axis=-1, keepdims=True)
    oh2 = iota == idx2
    denom = m1 + m2
    comb = jnp.where(oh1, m1 / denom, 0.0) + jnp.where(oh2, m2 / denom, 0.0)
    load = oh1.astype(jnp.float32) + oh2.astype(jnp.float32)

    @pl.when(i == 0)
    def _():
        accp_ref[...] = jnp.zeros_like(accp_ref)
        accl_ref[...] = jnp.zeros_like(accl_ref)

    accp_ref[...] += jnp.sum(probs, axis=0, keepdims=True)
    accl_ref[...] += jnp.sum(load, axis=0, keepdims=True)

    @pl.when(i == NB - 1)
    def _():
        aux = (NE / (N * N)) * jnp.sum(accp_ref[...] * accl_ref[...])
        aux_ref[...] = jnp.full((1, 1), aux, dtype=jnp.float32)

    act = _b(_gelu(ef))
    wup = _b(wup_ref[...])
    acc = jnp.zeros((T, D_MODEL), dtype=jnp.float32)
    for e in range(NE):
        z = jnp.dot(act[:, e * DL:(e + 1) * DL], wup[e * DL:(e + 1) * DL, :],
                    preferred_element_type=jnp.float32)
        acc = acc + jnp.broadcast_to(comb[:, e:e + 1], (T, D_MODEL)) * z
    out_ref[...] = acc


def kernel(x, w_down_W, expert_pos_embed, global_proj_W, global_proj_b,
           in_proj_W, in_proj_b, out_proj_W, out_proj_b, ln_gamma, ln_beta,
           scorer_W, scorer_b, mlp_W1, mlp_b1, mlp_W2, mlp_b2, w_up):
    x_flat = x.reshape(N, D_MODEL)

    ef, xsum = pl.pallas_call(
        _k1,
        grid=(NB,),
        in_specs=[
            pl.BlockSpec((T, D_MODEL), lambda i: (i, 0)),
            pl.BlockSpec((D_MODEL, NE * DL), lambda i: (0, 0)),
        ],
        out_specs=[
            pl.BlockSpec((T, NE * DL), lambda i: (i, 0)),
            pl.BlockSpec((1, 1, D_MODEL), lambda i: (i, 0, 0)),
        ],
        out_shape=[
            jax.ShapeDtypeStruct((N, NE * DL), jnp.float32),
            jax.ShapeDtypeStruct((NB, 1, D_MODEL), jnp.float32),
        ],
    )(x_flat, w_down_W.T)

    seg = np.zeros((L * DL, L * NH), dtype=np.float32)
    for j in range(L):
        for hh in range(NH):
            seg[j * DL + hh * HD:(j * DL + (hh + 1) * HD), j * NH + hh] = 0.25
    exp4 = np.zeros((NH, DL), dtype=np.float32)
    for hh in range(NH):
        exp4[hh, hh * HD:(hh + 1) * HD] = 1.0
    const2 = lambda i: (0, 0)
    small = lambda a: pl.BlockSpec(a.shape, const2)

    pos = expert_pos_embed.reshape(1, NE * DL)
    args = (
        ef, xsum, global_proj_W.T, pos,
        in_proj_W.T, in_proj_b.reshape(1, -1),
        out_proj_W.T, out_proj_b.reshape(1, -1),
        ln_gamma.reshape(1, -1), ln_beta.reshape(1, -1),
        scorer_W.reshape(-1, 1), scorer_b.reshape(1, 1),
        global_proj_b.reshape(1, -1),
        mlp_W1.T, mlp_b1.reshape(1, -1),
        mlp_W2.T, mlp_b2.reshape(1, -1),
        w_up.reshape(NE * DL, D_MODEL),
        jnp.asarray(seg), jnp.asarray(exp4),
    )

    in_specs = [
        pl.BlockSpec((T, NE * DL), lambda i: (i, 0)),
        pl.BlockSpec((NB, 1, D_MODEL), lambda i: (0, 0, 0)),
    ] + [small(a) for a in args[2:]]

    out, aux = pl.pallas_call(
        _k2,
        grid=(NB,),
        in_specs=in_specs,
        out_specs=[
            pl.BlockSpec((T, D_MODEL), lambda i: (i, 0)),
            pl.BlockSpec((1, 1), const2),
        ],
        out_shape=[
            jax.ShapeDtypeStruct((N, D_MODEL), jnp.float32),
            jax.ShapeDtypeStruct((1, 1), jnp.float32),
        ],
        scratch_shapes=[
            pltpu.VMEM((1, NE), jnp.float32),
            pltpu.VMEM((1, NE), jnp.float32),
        ],
    )(*args)

    return out.reshape(B, S, D_MODEL), aux[0, 0]
